# baseline (speedup 1.0000x reference)
import functools

import jax
import jax.numpy as jnp
from jax import lax
from jax.experimental import pallas as pl
from jax.experimental.pallas import tpu as pltpu
from jax.experimental.pallas import tpu_sc as plsc

N = 16384
M = 16
MAX_LENGTH = 20
TOKEN_DIM = 128
SPATIAL_DIM = 125
NW = 32
NSPLIT = 4
NSLICE = N // NSPLIT
PER_W = NSLICE // NW
CHUNK = 16
NCHUNK = PER_W // CHUNK
TAIL = TOKEN_DIM - 16


def _make_sc_slice(slice_base):
    mesh = plsc.VectorSubcoreMesh(core_axis_name="c", subcore_axis_name="s")

    @functools.partial(
        pl.kernel,
        mesh=mesh,
        out_type=jax.ShapeDtypeStruct((NSLICE, MAX_LENGTH, TOKEN_DIM), jnp.float32),
        scratch_types=[
            pltpu.VMEM((CHUNK, MAX_LENGTH, TOKEN_DIM), jnp.float32),
            pltpu.VMEM((CHUNK, MAX_LENGTH, TOKEN_DIM), jnp.float32),
            pltpu.VMEM((PER_W * M,), jnp.float32),
            pltpu.VMEM((PER_W,), jnp.float32),
            pltpu.VMEM((PER_W,), jnp.float32),
            pltpu.VMEM((M, SPATIAL_DIM), jnp.float32),
            pltpu.VMEM((M,), jnp.int32),
            pltpu.VMEM((16,), jnp.float32),
            pltpu.VMEM((16,), jnp.float32),
            pltpu.SemaphoreType.DMA,
            pltpu.SemaphoreType.DMA,
            pltpu.SemaphoreType.DMA,
            pltpu.SemaphoreType.DMA,
            pltpu.SemaphoreType.DMA,
        ],
    )
    def sc_kernel(values_hbm, tim_hbm, table_hbm, w_hbm, b_hbm, idx_hbm,
                  out_hbm, buf_a, buf_b, vals_v, tim0_v, tim1_v, rows_v,
                  idx_v, w_v, b_v, sem_a, sem_b, sem_in, sem_rows, sem_rep):
        wid = lax.axis_index("s") * 2 + lax.axis_index("c")
        base = wid * PER_W
        gbase = slice_base + wid * PER_W
        lane16 = lax.iota(jnp.int32, 16)

        cp_vals = pltpu.async_copy(
            values_hbm.at[pl.ds(gbase * M, PER_W * M)], vals_v, sem_in)
        cp_t0 = pltpu.async_copy(
            tim_hbm.at[0, pl.ds(gbase, PER_W)], tim0_v, sem_in)
        cp_t1 = pltpu.async_copy(
            tim_hbm.at[1, pl.ds(gbase, PER_W)], tim1_v, sem_in)
        cp_w = pltpu.async_copy(w_hbm, w_v, sem_in)
        cp_b = pltpu.async_copy(b_hbm, b_v, sem_in)
        pltpu.sync_copy(idx_hbm, idx_v)

        zero = jnp.zeros((16,), jnp.float32)
        for r in range(MAX_LENGTH):
            for c8 in range(TOKEN_DIM // 16):
                buf_a[0, r, pl.ds(c8 * 16, 16)] = zero

        iv = idx_v[...]
        row_copies = []
        for j in range(M):
            row_copies.append(
                pltpu.async_copy(
                    table_hbm.at[pl.ds(iv[j], 1), :],
                    rows_v.at[pl.ds(j, 1), :],
                    sem_rows,
                )
            )
        for c in row_copies:
            c.wait()
        for j in range(M):
            for c8 in range(SPATIAL_DIM // 16):
                buf_a[0, j, pl.ds(c8 * 16, 16)] = rows_v[j, pl.ds(c8 * 16, 16)]
            buf_a[0, j, pl.ds(SPATIAL_DIM - 16, 16)] = rows_v[
                j, pl.ds(SPATIAL_DIM - 16, 16)
            ]

        pltpu.sync_copy(buf_a.at[pl.ds(0, 1)], out_hbm.at[pl.ds(base, 1)])
        rep = []
        for s in range(1, CHUNK):
            rep.append(pltpu.async_copy(
                out_hbm.at[pl.ds(base, 1)], buf_a.at[pl.ds(s, 1)], sem_rep))
        for s in range(CHUNK):
            rep.append(pltpu.async_copy(
                out_hbm.at[pl.ds(base, 1)], buf_b.at[pl.ds(s, 1)], sem_rep))
        for c in rep:
            c.wait()

        cp_vals.wait()
        cp_t0.wait()
        cp_t1.wait()
        cp_w.wait()
        cp_b.wait()
        wv = w_v[...]
        bv = b_v[...]
        m_val = lane16 == (SPATIAL_DIM - TAIL)
        m_t0 = lane16 == (SPATIAL_DIM + 1 - TAIL)
        m_t1 = lane16 == (SPATIAL_DIM + 2 - TAIL)

        tails = [buf_a[0, j, pl.ds(TAIL, 16)] for j in range(M)]

        def insert(buf, chunk_idx):
            cbase = chunk_idx * CHUNK
            t0_all = tim0_v[pl.ds(cbase, CHUNK)]
            t1_all = tim1_v[pl.ds(cbase, CHUNK)]
            for t in range(CHUNK):
                valv = vals_v[pl.ds((cbase + t) * M, 16)] * wv + bv
                t0b = jnp.full((16,), t0_all[t], jnp.float32)
                t1b = jnp.full((16,), t1_all[t], jnp.float32)
                for j in range(M):
                    vj = jnp.full((16,), valv[j], jnp.float32)
                    merged = jnp.where(m_val, vj, tails[j])
                    merged = jnp.where(m_t0, t0b, merged)
                    merged = jnp.where(m_t1, t1b, merged)
                    buf[t, j, pl.ds(TAIL, 16)] = merged

        def fire(buf, chunk_idx, sem):
            pltpu.async_copy(
                buf, out_hbm.at[pl.ds(base + chunk_idx * CHUNK, CHUNK)], sem)

        def drain(buf, sem):
            pltpu.make_async_copy(
                buf, out_hbm.at[pl.ds(base, CHUNK)], sem).wait()

        def loop_body(i, _):
            @pl.when(i > 0)
            def _():
                drain(buf_a, sem_a)

            insert(buf_a, 2 * i)
            fire(buf_a, 2 * i, sem_a)

            @pl.when(i > 0)
            def _():
                drain(buf_b, sem_b)

            insert(buf_b, 2 * i + 1)
            fire(buf_b, 2 * i + 1, sem_b)
            return 0

        lax.fori_loop(0, NCHUNK // 2, loop_body, 0, unroll=False)
        drain(buf_a, sem_a)
        drain(buf_b, sem_b)

    return sc_kernel


def kernel(values, tim_emb, spatial_table, w_val, b_val, node_neighbors):
    w16 = jnp.broadcast_to(jnp.reshape(w_val, (1,)), (16,))
    b16 = jnp.broadcast_to(jnp.reshape(b_val, (1,)), (16,))
    vals_flat = values.reshape(-1)
    tim_t = tim_emb.T
    pieces = []
    for k in range(NSPLIT):
        sc_k = _make_sc_slice(k * NSLICE)
        pieces.append(
            sc_k(vals_flat, tim_t, spatial_table, w16, b16, node_neighbors)
        )
    return jnp.concatenate(pieces, axis=0)


# speedup vs baseline: 2.1799x; 2.1799x over previous
import functools

import jax
import jax.numpy as jnp
from jax import lax
from jax.experimental import pallas as pl
from jax.experimental.pallas import tpu as pltpu
from jax.experimental.pallas import tpu_sc as plsc

N = 16384
M = 16
MAX_LENGTH = 20
TOKEN_DIM = 128
SPATIAL_DIM = 125
NW = 32
PER_W = N // NW
CHUNK = 16
NCHUNK = PER_W // CHUNK
TAIL = TOKEN_DIM - 16


def kernel(values, tim_emb, spatial_table, w_val, b_val, node_neighbors):
    n = values.shape[0]
    w16 = jnp.broadcast_to(jnp.reshape(w_val, (1,)), (16,))
    b16 = jnp.broadcast_to(jnp.reshape(b_val, (1,)), (16,))
    tim_t = tim_emb.T
    mesh = plsc.VectorSubcoreMesh(core_axis_name="c", subcore_axis_name="s")

    @functools.partial(
        pl.kernel,
        mesh=mesh,
        out_type=jax.ShapeDtypeStruct((n, MAX_LENGTH, TOKEN_DIM), jnp.float32),
        scratch_types=[
            pltpu.VMEM((CHUNK, MAX_LENGTH, TOKEN_DIM), jnp.float32),
            pltpu.VMEM((CHUNK, MAX_LENGTH, TOKEN_DIM), jnp.float32),
            pltpu.VMEM((CHUNK, M), jnp.float32),
            pltpu.VMEM((CHUNK, M), jnp.float32),
            pltpu.VMEM((PER_W,), jnp.float32),
            pltpu.VMEM((PER_W,), jnp.float32),
            pltpu.VMEM((M, SPATIAL_DIM), jnp.float32),
            pltpu.VMEM((M,), jnp.int32),
            pltpu.VMEM((16,), jnp.float32),
            pltpu.VMEM((16,), jnp.float32),
            pltpu.SemaphoreType.DMA,
            pltpu.SemaphoreType.DMA,
            pltpu.SemaphoreType.DMA,
            pltpu.SemaphoreType.DMA,
            pltpu.SemaphoreType.DMA,
            pltpu.SemaphoreType.DMA,
            pltpu.SemaphoreType.DMA,
        ],
    )
    def sc_kernel(values_hbm, tim_hbm, table_hbm, w_hbm, b_hbm, idx_hbm,
                  out_hbm, buf_a, buf_b, vca, vcb, tim0_v, tim1_v, rows_v,
                  idx_v, w_v, b_v, sem_a, sem_b, sem_in, sem_rows, sem_rep,
                  sem_va, sem_vb):
        wid = lax.axis_index("s") * 2 + lax.axis_index("c")
        base = wid * PER_W
        lane16 = lax.iota(jnp.int32, 16)

        cp_t0 = pltpu.async_copy(
            tim_hbm.at[0, pl.ds(base, PER_W)], tim0_v, sem_in)
        cp_t1 = pltpu.async_copy(
            tim_hbm.at[1, pl.ds(base, PER_W)], tim1_v, sem_in)
        cp_w = pltpu.async_copy(w_hbm, w_v, sem_in)
        cp_b = pltpu.async_copy(b_hbm, b_v, sem_in)
        pltpu.sync_copy(idx_hbm, idx_v)

        def fetch_vals(vc, chunk_idx, sem):
            return pltpu.async_copy(
                values_hbm.at[pl.ds(base + chunk_idx * CHUNK, CHUNK), :],
                vc, sem)

        cva = fetch_vals(vca, 0, sem_va)
        cvb = fetch_vals(vcb, 1, sem_vb)

        zero = jnp.zeros((16,), jnp.float32)
        for r in range(MAX_LENGTH):
            for c8 in range(TOKEN_DIM // 16):
                buf_a[0, r, pl.ds(c8 * 16, 16)] = zero

        iv = idx_v[...]
        row_copies = []
        for j in range(M):
            row_copies.append(
                pltpu.async_copy(
                    table_hbm.at[pl.ds(iv[j], 1), :],
                    rows_v.at[pl.ds(j, 1), :],
                    sem_rows,
                )
            )
        for c in row_copies:
            c.wait()
        for j in range(M):
            for c8 in range(SPATIAL_DIM // 16):
                buf_a[0, j, pl.ds(c8 * 16, 16)] = rows_v[j, pl.ds(c8 * 16, 16)]
            buf_a[0, j, pl.ds(SPATIAL_DIM - 16, 16)] = rows_v[
                j, pl.ds(SPATIAL_DIM - 16, 16)
            ]

        pltpu.sync_copy(buf_a.at[pl.ds(0, 1)], out_hbm.at[pl.ds(base, 1)])
        rep = []
        for s in range(1, CHUNK):
            rep.append(pltpu.async_copy(
                out_hbm.at[pl.ds(base, 1)], buf_a.at[pl.ds(s, 1)], sem_rep))
        for s in range(CHUNK):
            rep.append(pltpu.async_copy(
                out_hbm.at[pl.ds(base, 1)], buf_b.at[pl.ds(s, 1)], sem_rep))
        for c in rep:
            c.wait()

        cp_t0.wait()
        cp_t1.wait()
        cp_w.wait()
        cp_b.wait()
        wv = w_v[...]
        bv = b_v[...]
        m_val = lane16 == (SPATIAL_DIM - TAIL)
        m_t0 = lane16 == (SPATIAL_DIM + 1 - TAIL)
        m_t1 = lane16 == (SPATIAL_DIM + 2 - TAIL)

        tails = [buf_a[0, j, pl.ds(TAIL, 16)] for j in range(M)]

        def vals_wait(vc, sem):
            pltpu.make_async_copy(
                values_hbm.at[pl.ds(0, CHUNK), :], vc, sem).wait()

        def insert(buf, vc, chunk_idx):
            cbase = chunk_idx * CHUNK
            t0_all = tim0_v[pl.ds(cbase, CHUNK)]
            t1_all = tim1_v[pl.ds(cbase, CHUNK)]
            for t in range(CHUNK):
                valv = vc[t, pl.ds(0, M)] * wv + bv
                t0b = jnp.full((16,), t0_all[t], jnp.float32)
                t1b = jnp.full((16,), t1_all[t], jnp.float32)
                for j in range(M):
                    vj = jnp.full((16,), valv[j], jnp.float32)
                    merged = jnp.where(m_val, vj, tails[j])
                    merged = jnp.where(m_t0, t0b, merged)
                    merged = jnp.where(m_t1, t1b, merged)
                    buf[t, j, pl.ds(TAIL, 16)] = merged

        def fire(buf, chunk_idx, sem):
            pltpu.async_copy(
                buf, out_hbm.at[pl.ds(base + chunk_idx * CHUNK, CHUNK)], sem)

        def drain(buf, sem):
            pltpu.make_async_copy(
                buf, out_hbm.at[pl.ds(base, CHUNK)], sem).wait()

        def loop_body(i, _):
            @pl.when(i > 0)
            def _():
                drain(buf_a, sem_a)

            vals_wait(vca, sem_va)
            insert(buf_a, vca, 2 * i)
            fire(buf_a, 2 * i, sem_a)

            @pl.when(2 * i + 2 < NCHUNK)
            def _():
                fetch_vals(vca, 2 * i + 2, sem_va)

            @pl.when(i > 0)
            def _():
                drain(buf_b, sem_b)

            vals_wait(vcb, sem_vb)
            insert(buf_b, vcb, 2 * i + 1)
            fire(buf_b, 2 * i + 1, sem_b)

            @pl.when(2 * i + 3 < NCHUNK)
            def _():
                fetch_vals(vcb, 2 * i + 3, sem_vb)
            return 0

        lax.fori_loop(0, NCHUNK // 2, loop_body, 0, unroll=False)
        drain(buf_a, sem_a)
        drain(buf_b, sem_b)

    return sc_kernel(values, tim_t, spatial_table, w16, b16, node_neighbors)
